# baseline (speedup 1.0000x reference)
import functools

import jax
import jax.numpy as jnp
from jax import lax
from jax.experimental import pallas as pl
from jax.experimental.pallas import tpu as pltpu
from jax.experimental.pallas import tpu_sc as plsc

N = 320000
D = 128
NUM_FINE = 100
NUM_SUPER = 20
FINE_PER_SUPER = 5

NC = 2
NS = 16
LANES = 16
NW = NC * NS
N_SC = 166400
N_TC = N - N_SC
PER_W = N_SC // NW
TC_B = 1280
TC_G = N_TC // TC_B
CHUNK = 400
N_CHUNKS = PER_W // CHUNK
SUB = 80
NSUB = CHUNK // SUB


def _sc_partials_body(emb_hbm, lbl_hbm, sums_hbm, cnts_hbm, emb_a, emb_b,
                      lbl_a, lbl_b, zero_c, acc_s, acc_c, sem_a, sem_b,
                      sem_sc):
    cid = lax.axis_index("c")
    sid = lax.axis_index("s")
    wid = sid * NC + cid
    base = wid * PER_W

    zeros16 = jnp.zeros((LANES,), jnp.float32)
    ones16 = jnp.ones((LANES,), jnp.float32)

    @pl.when(sid == 0)
    def _():
        def zero_s_body(r, _):
            for j in range(D // LANES):
                emb_a[r, pl.ds(j * LANES, LANES)] = zeros16
            return _

        lax.fori_loop(0, NUM_FINE, zero_s_body, None)
        pltpu.sync_copy(emb_a.at[pl.ds(0, NUM_FINE)], acc_s)

    def zero_c_body(r, _):
        acc_c[r, :] = zeros16
        return _

    lax.fori_loop(0, NUM_FINE, zero_c_body, None)
    plsc.subcore_barrier()

    def start_fetch(c, emb_v, lbl_v, sem):
        start = base + c * CHUNK
        pltpu.async_copy(emb_hbm.at[pl.ds(start, CHUNK)], emb_v, sem)
        for s in range(NSUB):
            pltpu.async_copy(lbl_hbm.at[pl.ds(start + s * SUB, SUB)],
                             lbl_v.at[s], sem)

    def wait_fetch(emb_v, lbl_v, sem):
        pltpu.make_async_copy(emb_hbm.at[pl.ds(0, CHUNK)], emb_v, sem).wait()
        for s in range(NSUB):
            pltpu.make_async_copy(lbl_hbm.at[pl.ds(0, SUB)], lbl_v.at[s],
                                  sem).wait()

    def accumulate(emb_v, lbl_v):
        descs = []
        for s in range(NSUB):
            descs.append(pltpu.async_copy(emb_v.at[pl.ds(s * SUB, SUB)],
                                          acc_s.at[lbl_v.at[s]], sem_sc,
                                          add=True))

        def cnt_body(s, _):
            for g in range(SUB // LANES):
                lblv = lbl_v[s, pl.ds(g * LANES, LANES)]
                for r in range(LANES):
                    plsc.addupdate(acc_c.at[lblv[r], :], ones16)
            return _

        lax.fori_loop(0, NSUB, cnt_body, None)

        for desc in descs:
            desc.wait()

    start_fetch(0, emb_a, lbl_a, sem_a)

    def pair_body(c2, _):
        c = 2 * c2
        start_fetch(c + 1, emb_b, lbl_b, sem_b)
        wait_fetch(emb_a, lbl_a, sem_a)
        accumulate(emb_a, lbl_a)
        start_fetch(c + 2, emb_a, lbl_a, sem_a)
        wait_fetch(emb_b, lbl_b, sem_b)
        accumulate(emb_b, lbl_b)
        return _

    lax.fori_loop(0, (N_CHUNKS - 1) // 2, pair_body, None)

    wait_fetch(emb_a, lbl_a, sem_a)
    accumulate(emb_a, lbl_a)

    pltpu.sync_copy(acc_c, cnts_hbm.at[wid])
    plsc.subcore_barrier()

    @pl.when(sid == 0)
    def _():
        pltpu.sync_copy(acc_s, sums_hbm.at[cid])


def _sc_partials(embeddings, labels):
    mesh = plsc.VectorSubcoreMesh(core_axis_name="c", subcore_axis_name="s",
                                  num_cores=NC, num_subcores=NS)
    return pl.kernel(
        _sc_partials_body,
        out_type=(
            jax.ShapeDtypeStruct((NC, NUM_FINE, D), jnp.float32),
            jax.ShapeDtypeStruct((NW, NUM_FINE, LANES), jnp.float32),
        ),
        mesh=mesh,
        scratch_types=[
            pltpu.VMEM((CHUNK, D), jnp.float32),
            pltpu.VMEM((CHUNK, D), jnp.float32),
            pltpu.VMEM((NSUB, SUB), jnp.int32),
            pltpu.VMEM((NSUB, SUB), jnp.int32),
            pltpu.VMEM((NUM_FINE, LANES), jnp.float32),
            pltpu.VMEM_SHARED((NUM_FINE, D), jnp.float32),
            pltpu.VMEM((NUM_FINE, LANES), jnp.float32),
            pltpu.SemaphoreType.DMA,
            pltpu.SemaphoreType.DMA,
            pltpu.SemaphoreType.DMA,
        ],
    )(embeddings, labels)


def _tc_partials_body(lbl_ref, emb_ref, sums_ref, cnts_ref):
    @pl.when(pl.program_id(0) == 0)
    def _():
        sums_ref[...] = jnp.zeros_like(sums_ref)
        cnts_ref[...] = jnp.zeros_like(cnts_ref)

    iota_c = lax.broadcasted_iota(jnp.int32, (NUM_FINE, TC_B), 0)
    lbl = lbl_ref[...].reshape(1, TC_B)
    oh = (lbl == iota_c).astype(jnp.float32)
    sums_ref[...] += jax.lax.dot_general(
        oh, emb_ref[...], (((1,), (0,)), ((), ())),
        preferred_element_type=jnp.float32)
    cnts_ref[...] += jnp.sum(oh, axis=1, keepdims=True)


def _tc_partials(embeddings, labels2d):
    return pl.pallas_call(
        _tc_partials_body,
        grid=(TC_G,),
        in_specs=[
            pl.BlockSpec((1, 1, TC_B), lambda g: (N_SC // TC_B + g, 0, 0)),
            pl.BlockSpec((TC_B, D), lambda g: (N_SC // TC_B + g, 0)),
        ],
        out_specs=[
            pl.BlockSpec((NUM_FINE, D), lambda g: (0, 0)),
            pl.BlockSpec((NUM_FINE, 1), lambda g: (0, 0)),
        ],
        out_shape=[
            jax.ShapeDtypeStruct((NUM_FINE, D), jnp.float32),
            jax.ShapeDtypeStruct((NUM_FINE, 1), jnp.float32),
        ],
    )(labels2d, embeddings)


def _loss_body(sums_ref, cnts_ref, tcs_ref, tcc_ref, ref_fine_ref,
               ref_super_ref, ref_inter_ref, out_ref):
    sums = jnp.sum(sums_ref[...], axis=0) + tcs_ref[...]
    counts = jnp.sum(cnts_ref[...], axis=0)[:, 0] + tcc_ref[..., 0]

    fine_cent = sums / jnp.maximum(counts, 1.0)[:, None]
    fine_present = (counts > 0).astype(jnp.float32)
    fine_err = jnp.mean((fine_cent - ref_fine_ref[...]) ** 2, axis=1)
    fine_loss = jnp.sum(fine_present * fine_err)

    super_sums = jnp.sum(sums.reshape(NUM_SUPER, FINE_PER_SUPER, D), axis=1)
    super_counts = jnp.sum(counts.reshape(NUM_SUPER, FINE_PER_SUPER), axis=1)
    super_cent = super_sums / jnp.maximum(super_counts, 1.0)[:, None]
    super_present = (super_counts > 0).astype(jnp.float32)
    super_err = jnp.mean((super_cent - ref_super_ref[...]) ** 2, axis=1)
    super_loss = jnp.sum(super_present * super_err)

    d = super_cent[:, None, :] - super_cent[None, :, :]
    cur_dist = jnp.sqrt(jnp.sum(d * d, axis=-1) + 1e-12)
    row = lax.broadcasted_iota(jnp.int32, (NUM_SUPER, NUM_SUPER), 0)
    col = lax.broadcasted_iota(jnp.int32, (NUM_SUPER, NUM_SUPER), 1)
    pair_mask = ((col > row).astype(jnp.float32)
                 * super_present[:, None] * super_present[None, :])
    inter_loss = jnp.sum(pair_mask * (cur_dist - ref_inter_ref[...]) ** 2)

    out_ref[...] = jnp.reshape(fine_loss + super_loss + inter_loss, (1, 1))


def _loss(sums, cnts, tcs, tcc, ref_fine, ref_super, ref_inter):
    out = pl.pallas_call(
        _loss_body,
        out_shape=jax.ShapeDtypeStruct((1, 1), jnp.float32),
    )(sums, cnts, tcs, tcc, ref_fine, ref_super, ref_inter)
    return out[0, 0]


@jax.jit
def _run(embeddings, labels, labels2d, ref_fine, ref_super, ref_inter):
    sums, cnts = _sc_partials(embeddings, labels)
    tcs, tcc = _tc_partials(embeddings, labels2d)
    return _loss(sums, cnts, tcs, tcc, ref_fine, ref_super, ref_inter)


def kernel(embeddings, labels, ref_fine, ref_super, ref_inter):
    labels = labels.astype(jnp.int32)
    labels2d = labels.reshape(N // TC_B, 1, TC_B)
    return _run(embeddings, labels, labels2d, ref_fine, ref_super, ref_inter)


# speedup vs baseline: 1.2519x; 1.2519x over previous
import functools

import jax
import jax.numpy as jnp
from jax import lax
from jax.experimental import pallas as pl
from jax.experimental.pallas import tpu as pltpu
from jax.experimental.pallas import tpu_sc as plsc

N = 320000
D = 128
NUM_FINE = 100
NUM_SUPER = 20
FINE_PER_SUPER = 5

NC = 2
NS = 16
LANES = 16
NW = NC * NS
N_SC = 217600
N_TC = N - N_SC
PER_W = N_SC // NW
TC_B = 1280
TC_G = N_TC // TC_B
CHUNK = 400
N_CHUNKS = PER_W // CHUNK
SUB = 80
NSUB = CHUNK // SUB


def _sc_partials_body(emb_hbm, lbl_hbm, sums_hbm, cnts_hbm, emb_a, emb_b,
                      lbl_a, lbl_b, zero_c, acc_s, acc_c, sem_a, sem_b,
                      sem_sc):
    cid = lax.axis_index("c")
    sid = lax.axis_index("s")
    wid = sid * NC + cid
    base = wid * PER_W

    zeros16 = jnp.zeros((LANES,), jnp.float32)
    ones16 = jnp.ones((LANES,), jnp.float32)

    @pl.when(sid == 0)
    def _():
        def zero_s_body(r, _):
            for j in range(D // LANES):
                emb_a[r, pl.ds(j * LANES, LANES)] = zeros16
            return _

        lax.fori_loop(0, NUM_FINE, zero_s_body, None)
        pltpu.sync_copy(emb_a.at[pl.ds(0, NUM_FINE)], acc_s)

    def zero_c_body(r, _):
        acc_c[r, :] = zeros16
        return _

    lax.fori_loop(0, NUM_FINE, zero_c_body, None)
    plsc.subcore_barrier()

    def start_fetch(c, emb_v, lbl_v, sem):
        start = base + c * CHUNK
        pltpu.async_copy(emb_hbm.at[pl.ds(start, CHUNK)], emb_v, sem)
        for s in range(NSUB):
            pltpu.async_copy(lbl_hbm.at[pl.ds(start + s * SUB, SUB)],
                             lbl_v.at[s], sem)

    def wait_fetch(emb_v, lbl_v, sem):
        pltpu.make_async_copy(emb_hbm.at[pl.ds(0, CHUNK)], emb_v, sem).wait()
        for s in range(NSUB):
            pltpu.make_async_copy(lbl_hbm.at[pl.ds(0, SUB)], lbl_v.at[s],
                                  sem).wait()

    def accumulate(emb_v, lbl_v):
        descs = []
        for s in range(NSUB):
            descs.append(pltpu.async_copy(emb_v.at[pl.ds(s * SUB, SUB)],
                                          acc_s.at[lbl_v.at[s]], sem_sc,
                                          add=True))

        def cnt_body(s, _):
            for g in range(SUB // LANES):
                lblv = lbl_v[s, pl.ds(g * LANES, LANES)]
                for r in range(LANES):
                    plsc.addupdate(acc_c.at[lblv[r], :], ones16)
            return _

        lax.fori_loop(0, NSUB, cnt_body, None)

        for desc in descs:
            desc.wait()

    start_fetch(0, emb_a, lbl_a, sem_a)

    def pair_body(c2, _):
        c = 2 * c2
        start_fetch(c + 1, emb_b, lbl_b, sem_b)
        wait_fetch(emb_a, lbl_a, sem_a)
        accumulate(emb_a, lbl_a)
        start_fetch(c + 2, emb_a, lbl_a, sem_a)
        wait_fetch(emb_b, lbl_b, sem_b)
        accumulate(emb_b, lbl_b)
        return _

    lax.fori_loop(0, (N_CHUNKS - 1) // 2, pair_body, None)

    wait_fetch(emb_a, lbl_a, sem_a)
    accumulate(emb_a, lbl_a)

    pltpu.sync_copy(acc_c, cnts_hbm.at[wid])
    plsc.subcore_barrier()

    @pl.when(sid == 0)
    def _():
        pltpu.sync_copy(acc_s, sums_hbm.at[cid])


def _sc_partials(embeddings, labels):
    mesh = plsc.VectorSubcoreMesh(core_axis_name="c", subcore_axis_name="s",
                                  num_cores=NC, num_subcores=NS)
    return pl.kernel(
        _sc_partials_body,
        out_type=(
            jax.ShapeDtypeStruct((NC, NUM_FINE, D), jnp.float32),
            jax.ShapeDtypeStruct((NW, NUM_FINE, LANES), jnp.float32),
        ),
        mesh=mesh,
        scratch_types=[
            pltpu.VMEM((CHUNK, D), jnp.float32),
            pltpu.VMEM((CHUNK, D), jnp.float32),
            pltpu.VMEM((NSUB, SUB), jnp.int32),
            pltpu.VMEM((NSUB, SUB), jnp.int32),
            pltpu.VMEM((NUM_FINE, LANES), jnp.float32),
            pltpu.VMEM_SHARED((NUM_FINE, D), jnp.float32),
            pltpu.VMEM((NUM_FINE, LANES), jnp.float32),
            pltpu.SemaphoreType.DMA,
            pltpu.SemaphoreType.DMA,
            pltpu.SemaphoreType.DMA,
        ],
    )(embeddings, labels)


def _tc_partials_body(lbl_ref, emb_ref, sums_ref, cnts_ref):
    @pl.when(pl.program_id(0) == 0)
    def _():
        sums_ref[...] = jnp.zeros_like(sums_ref)
        cnts_ref[...] = jnp.zeros_like(cnts_ref)

    iota_c = lax.broadcasted_iota(jnp.int32, (NUM_FINE, TC_B), 0)
    lbl = lbl_ref[...].reshape(1, TC_B)
    oh = (lbl == iota_c).astype(jnp.float32)
    sums_ref[...] += jax.lax.dot_general(
        oh, emb_ref[...], (((1,), (0,)), ((), ())),
        preferred_element_type=jnp.float32)
    cnts_ref[...] += jnp.sum(oh, axis=1, keepdims=True)


def _tc_partials(embeddings, labels2d):
    return pl.pallas_call(
        _tc_partials_body,
        grid=(TC_G,),
        in_specs=[
            pl.BlockSpec((1, 1, TC_B), lambda g: (N_SC // TC_B + g, 0, 0)),
            pl.BlockSpec((TC_B, D), lambda g: (N_SC // TC_B + g, 0)),
        ],
        out_specs=[
            pl.BlockSpec((NUM_FINE, D), lambda g: (0, 0)),
            pl.BlockSpec((NUM_FINE, 1), lambda g: (0, 0)),
        ],
        out_shape=[
            jax.ShapeDtypeStruct((NUM_FINE, D), jnp.float32),
            jax.ShapeDtypeStruct((NUM_FINE, 1), jnp.float32),
        ],
    )(labels2d, embeddings)


def _loss_body(sums_ref, cnts_ref, tcs_ref, tcc_ref, ref_fine_ref,
               ref_super_ref, ref_inter_ref, out_ref):
    sums = jnp.sum(sums_ref[...], axis=0) + tcs_ref[...]
    counts = jnp.sum(cnts_ref[...], axis=0)[:, 0] + tcc_ref[..., 0]

    fine_cent = sums / jnp.maximum(counts, 1.0)[:, None]
    fine_present = (counts > 0).astype(jnp.float32)
    fine_err = jnp.mean((fine_cent - ref_fine_ref[...]) ** 2, axis=1)
    fine_loss = jnp.sum(fine_present * fine_err)

    super_sums = jnp.sum(sums.reshape(NUM_SUPER, FINE_PER_SUPER, D), axis=1)
    super_counts = jnp.sum(counts.reshape(NUM_SUPER, FINE_PER_SUPER), axis=1)
    super_cent = super_sums / jnp.maximum(super_counts, 1.0)[:, None]
    super_present = (super_counts > 0).astype(jnp.float32)
    super_err = jnp.mean((super_cent - ref_super_ref[...]) ** 2, axis=1)
    super_loss = jnp.sum(super_present * super_err)

    d = super_cent[:, None, :] - super_cent[None, :, :]
    cur_dist = jnp.sqrt(jnp.sum(d * d, axis=-1) + 1e-12)
    row = lax.broadcasted_iota(jnp.int32, (NUM_SUPER, NUM_SUPER), 0)
    col = lax.broadcasted_iota(jnp.int32, (NUM_SUPER, NUM_SUPER), 1)
    pair_mask = ((col > row).astype(jnp.float32)
                 * super_present[:, None] * super_present[None, :])
    inter_loss = jnp.sum(pair_mask * (cur_dist - ref_inter_ref[...]) ** 2)

    out_ref[...] = jnp.reshape(fine_loss + super_loss + inter_loss, (1, 1))


def _loss(sums, cnts, tcs, tcc, ref_fine, ref_super, ref_inter):
    out = pl.pallas_call(
        _loss_body,
        out_shape=jax.ShapeDtypeStruct((1, 1), jnp.float32),
    )(sums, cnts, tcs, tcc, ref_fine, ref_super, ref_inter)
    return out[0, 0]


@jax.jit
def _run(embeddings, labels, labels2d, ref_fine, ref_super, ref_inter):
    sums, cnts = _sc_partials(embeddings, labels)
    tcs, tcc = _tc_partials(embeddings, labels2d)
    return _loss(sums, cnts, tcs, tcc, ref_fine, ref_super, ref_inter)


def kernel(embeddings, labels, ref_fine, ref_super, ref_inter):
    labels = labels.astype(jnp.int32)
    labels2d = labels.reshape(N // TC_B, 1, TC_B)
    return _run(embeddings, labels, labels2d, ref_fine, ref_super, ref_inter)
